# Initial kernel scaffold; baseline (speedup 1.0000x reference)
import jax
import jax.numpy as jnp
from jax.experimental import pallas as pl


def kernel(objectness, box_regression, anchors_bbox3d):
    raise NotImplementedError("write your pallas kernel here")


# speedup vs baseline: 3.5867x; 3.5867x over previous
import jax
import jax.numpy as jnp
from jax.experimental import pallas as pl

_PRE_NMS_TOP_N = 2000
_POST_NMS_TOP_N = 500
_NMS_THRESH = 0.7


def _decode_nms_kernel(scores_ref, deltas_ref, anchors_ref, props_ref, keep_ref):
    an = anchors_ref[...]
    dl = deltas_ref[...]
    xa, ya, za = an[0:1], an[1:2], an[2:3]
    wa, la, ha = an[3:4], an[4:5], an[5:6]
    ra = an[6:7]
    xt, yt, zt = dl[0:1], dl[1:2], dl[2:3]
    wt, lt, ht = dl[3:4], dl[4:5], dl[5:6]
    rt = dl[6:7]
    diag = jnp.sqrt(wa * wa + la * la)
    x = xt * diag + xa
    y = yt * diag + ya
    z = zt * ha + za
    w = jnp.exp(wt) * wa
    l = jnp.exp(lt) * la
    h = jnp.exp(ht) * ha
    r = rt + ra
    props_ref[...] = jnp.concatenate([x, y, z, w, l, h, r], axis=0)

    x1 = x - w * 0.5
    x2 = x + w * 0.5
    y1 = y - l * 0.5
    y2 = y + l * 0.5
    area = w * l
    iota = jax.lax.broadcasted_iota(jnp.int32, x.shape, 1)

    def body(i, keep):
        m = iota == i

        def sel(v):
            return jnp.sum(jnp.where(m, v, 0.0))

        xi1 = sel(x1)
        xi2 = sel(x2)
        yi1 = sel(y1)
        yi2 = sel(y2)
        ai = sel(area)
        ki = jnp.sum(jnp.where(m, keep, 0.0))
        iw = jnp.clip(jnp.minimum(xi2, x2) - jnp.maximum(xi1, x1), 0.0)
        ih = jnp.clip(jnp.minimum(yi2, y2) - jnp.maximum(yi1, y1), 0.0)
        inter = iw * ih
        iou = inter / (ai + area - inter + 1e-9)
        sup = (iou > _NMS_THRESH) & (iota > i) & (ki > 0.5)
        return jnp.where(sup, 0.0, keep)

    keep = jax.lax.fori_loop(0, x.shape[1], body,
                             jnp.ones(x.shape, dtype=jnp.float32))
    keep_ref[...] = keep
    del scores_ref


def kernel(objectness, box_regression, anchors_bbox3d):
    N, A, H, W = objectness.shape
    obj = jax.nn.sigmoid(jnp.transpose(objectness, (0, 2, 3, 1)).reshape(N, -1))
    breg = jnp.transpose(box_regression.reshape(N, A, 7, H, W),
                         (0, 3, 4, 1, 2)).reshape(N, -1, 7)
    num_anchors = A * H * W
    pre = min(_PRE_NMS_TOP_N, num_anchors)
    scores, topk_idx = jax.lax.top_k(obj, pre)
    deltas_t = jnp.take(breg[0], topk_idx[0], axis=0).T
    anchors_t = jnp.take(anchors_bbox3d.reshape(-1, 7), topk_idx[0], axis=0).T

    props, keep = pl.pallas_call(
        _decode_nms_kernel,
        out_shape=[
            jax.ShapeDtypeStruct((7, pre), jnp.float32),
            jax.ShapeDtypeStruct((1, pre), jnp.float32),
        ],
    )(scores, deltas_t, anchors_t)

    scores = scores[0]
    masked = jnp.where(keep[0] > 0.5, scores, -1e9)
    post = min(_POST_NMS_TOP_N, pre)
    _, inds = jax.lax.top_k(masked, post)
    boxes_out = jnp.take(props.T, inds, axis=0)
    scores_out = jnp.take(scores, inds, axis=0)
    return jnp.concatenate([boxes_out, scores_out[:, None]], axis=1)


# speedup vs baseline: 3.7739x; 1.0522x over previous
import jax
import jax.numpy as jnp
from jax.experimental import pallas as pl

_PRE_NMS_TOP_N = 2000
_POST_NMS_TOP_N = 500
_NMS_THRESH = 0.7


def _decode_nms_kernel(scores_ref, deltas_ref, anchors_ref, props_ref, keep_ref):
    an = anchors_ref[...]
    dl = deltas_ref[...]
    xa, ya, za = an[0:1], an[1:2], an[2:3]
    wa, la, ha = an[3:4], an[4:5], an[5:6]
    ra = an[6:7]
    xt, yt, zt = dl[0:1], dl[1:2], dl[2:3]
    wt, lt, ht = dl[3:4], dl[4:5], dl[5:6]
    rt = dl[6:7]
    diag = jnp.sqrt(wa * wa + la * la)
    x = xt * diag + xa
    y = yt * diag + ya
    z = zt * ha + za
    w = jnp.exp(wt) * wa
    l = jnp.exp(lt) * la
    h = jnp.exp(ht) * ha
    r = rt + ra
    props_ref[...] = jnp.concatenate([x, y, z, w, l, h, r], axis=0)

    x1 = x - w * 0.5
    x2 = x + w * 0.5
    y1 = y - l * 0.5
    y2 = y + l * 0.5
    area = w * l
    iota = jax.lax.broadcasted_iota(jnp.int32, x.shape, 1)
    packed = jnp.concatenate([x1, x2, y1, y2, area], axis=0)

    def body(i, keep):
        m = iota == i
        s = jnp.sum(jnp.where(m, packed, 0.0), axis=1, keepdims=True)
        xi1 = s[0:1]
        xi2 = s[1:2]
        yi1 = s[2:3]
        yi2 = s[3:4]
        ai = s[4:5]
        ki = jnp.sum(jnp.where(m, keep, 0.0))
        iw = jnp.clip(jnp.minimum(xi2, x2) - jnp.maximum(xi1, x1), 0.0)
        ih = jnp.clip(jnp.minimum(yi2, y2) - jnp.maximum(yi1, y1), 0.0)
        inter = iw * ih
        iou = inter / (ai + area - inter + 1e-9)
        sup = (iou > _NMS_THRESH) & (iota > i) & (ki > 0.5)
        return jnp.where(sup, 0.0, keep)

    keep = jax.lax.fori_loop(0, x.shape[1], body,
                             jnp.ones(x.shape, dtype=jnp.float32))
    keep_ref[...] = keep
    del scores_ref


def kernel(objectness, box_regression, anchors_bbox3d):
    N, A, H, W = objectness.shape
    obj = jax.nn.sigmoid(jnp.transpose(objectness, (0, 2, 3, 1)).reshape(N, -1))
    breg = jnp.transpose(box_regression.reshape(N, A, 7, H, W),
                         (0, 3, 4, 1, 2)).reshape(N, -1, 7)
    num_anchors = A * H * W
    pre = min(_PRE_NMS_TOP_N, num_anchors)
    scores, topk_idx = jax.lax.top_k(obj, pre)
    deltas_t = jnp.take(breg[0], topk_idx[0], axis=0).T
    anchors_t = jnp.take(anchors_bbox3d.reshape(-1, 7), topk_idx[0], axis=0).T

    props, keep = pl.pallas_call(
        _decode_nms_kernel,
        out_shape=[
            jax.ShapeDtypeStruct((7, pre), jnp.float32),
            jax.ShapeDtypeStruct((1, pre), jnp.float32),
        ],
    )(scores, deltas_t, anchors_t)

    scores = scores[0]
    masked = jnp.where(keep[0] > 0.5, scores, -1e9)
    post = min(_POST_NMS_TOP_N, pre)
    _, inds = jax.lax.top_k(masked, post)
    boxes_out = jnp.take(props.T, inds, axis=0)
    scores_out = jnp.take(scores, inds, axis=0)
    return jnp.concatenate([boxes_out, scores_out[:, None]], axis=1)
